# baseline (speedup 1.0000x reference)
import functools

import jax
import jax.numpy as jnp
from jax import lax
from jax.experimental import pallas as pl
from jax.experimental.pallas import tpu as pltpu
from jax.experimental.pallas import tpu_sc as plsc

B, S = 4, 8192
D = 1024
NT = 8
N = B * S
NH = N
NC, NS = 2, 16
NW = NC * NS
TOK_PER_W = NH // NW
C = 32
NCHUNK = TOK_PER_W // C

_mesh = plsc.VectorSubcoreMesh(core_axis_name="c", subcore_axis_name="s")


@functools.partial(
    pl.kernel,
    mesh=_mesh,
    compiler_params=pltpu.CompilerParams(needs_layout_passes=False),
    out_type=jax.ShapeDtypeStruct((NH, D), jnp.float32),
    scratch_types=[
        pltpu.VMEM((TOK_PER_W + 2 * C,), jnp.int32),
        pltpu.VMEM((C, D), jnp.float32),
        pltpu.VMEM((C, D), jnp.float32),
        pltpu.VMEM((C, D), jnp.float32),
        pltpu.SemaphoreType.DMA,
        pltpu.SemaphoreType.DMA,
        pltpu.SemaphoreType.DMA,
        pltpu.SemaphoreType.DMA,
        pltpu.SemaphoreType.DMA,
        pltpu.SemaphoreType.DMA,
    ],
)
def _gather_sc(ids_hbm, tok_tbl_hbm, out_hbm,
               idx_all, tok0, tok1, tok2,
               gsem0, gsem1, gsem2, osem0, osem1, osem2):
    wid = lax.axis_index("s") * NC + lax.axis_index("c")
    base = wid * TOK_PER_W
    toks = (tok0, tok1, tok2)
    gsems = (gsem0, gsem1, gsem2)
    osems = (osem0, osem1, osem2)

    pltpu.sync_copy(ids_hbm.at[pl.ds(base, TOK_PER_W)],
                    idx_all.at[pl.ds(0, TOK_PER_W)])
    for p in range(2 * C // 16):
        idx_all[pl.ds(TOK_PER_W + p * 16, 16)] = jnp.zeros((16,), jnp.int32)

    def start_gather(c, b):
        pltpu.async_copy(tok_tbl_hbm.at[idx_all.at[pl.ds(c * C, C)]],
                         toks[b], gsems[b])

    def wait_gather(b):
        pltpu.make_async_copy(tok_tbl_hbm.at[pl.ds(0, C)], toks[b],
                              gsems[b]).wait()

    def start_out(c, b):
        pltpu.async_copy(toks[b], out_hbm.at[pl.ds(base + c * C, C)],
                         osems[b])

    def wait_out(b):
        pltpu.make_async_copy(toks[b], out_hbm.at[pl.ds(0, C)],
                              osems[b]).wait()

    start_gather(0, 0)
    start_gather(1, 1)

    def _phase(c, carry):
        par = lax.rem(c, 3)

        def body(b):
            nb = (b + 2) % 3
            @pl.when(c > 0)
            def _():
                wait_out(nb)

            start_gather(c + 2, nb)
            wait_gather(b)
            start_out(c, b)

        for bb in range(3):
            @pl.when(par == bb)
            def _(bb=bb):
                body(bb)

        return carry

    lax.fori_loop(0, NCHUNK, _phase, 0)

    wait_gather(NCHUNK % 3)
    wait_gather((NCHUNK + 1) % 3)
    wait_out((NCHUNK - 1) % 3)


BLK = 2048


def _add_body(cids_ref, tok_ref, ctbl_ref, out_ref):
    cid = cids_ref[0, 0, :]
    onehot = (cid[:, None] == lax.iota(jnp.int32, NT)[None, :])
    code = jnp.dot(onehot.astype(jnp.float32), ctbl_ref[...],
                   preferred_element_type=jnp.float32)
    out_ref[...] = tok_ref[...] + code


_add_tc = pl.pallas_call(
    _add_body,
    grid=(NH // BLK,),
    in_specs=[
        pl.BlockSpec((1, 1, BLK), lambda i: (i, 0, 0)),
        pl.BlockSpec((BLK, D), lambda i: (i, 0)),
        pl.BlockSpec((NT, D), lambda i: (0, 0)),
    ],
    out_specs=pl.BlockSpec((BLK, D), lambda i: (i, 0)),
    out_shape=jax.ShapeDtypeStruct((NH, D), jnp.float32),
)


def kernel(input_ids, code_type_ids, token_table, code_table):
    ids = input_ids.reshape(N).astype(jnp.int32)
    cids = code_type_ids.reshape(N // BLK, 1, BLK).astype(jnp.int32)
    g = _gather_sc(ids, token_table)
    return _add_tc(cids, g, code_table).reshape(B, S, D)


# speedup vs baseline: 1.2173x; 1.2173x over previous
import functools

import jax
import jax.numpy as jnp
from jax import lax
from jax.experimental import pallas as pl
from jax.experimental.pallas import tpu as pltpu
from jax.experimental.pallas import tpu_sc as plsc

B, S = 4, 8192
D = 1024
NT = 8
N = B * S
NH = N
NC, NS = 2, 16
NW = NC * NS
TOK_PER_W = NH // NW
C = 32
NCHUNK = TOK_PER_W // C

_mesh = plsc.VectorSubcoreMesh(core_axis_name="c", subcore_axis_name="s")


@functools.partial(
    pl.kernel,
    mesh=_mesh,
    compiler_params=pltpu.CompilerParams(needs_layout_passes=False),
    out_type=jax.ShapeDtypeStruct((NH, D), jnp.float32),
    scratch_types=[
        pltpu.VMEM((TOK_PER_W + C,), jnp.int32),
        pltpu.VMEM((C, D), jnp.float32),
        pltpu.VMEM((C, D), jnp.float32),
        pltpu.SemaphoreType.DMA,
        pltpu.SemaphoreType.DMA,
        pltpu.SemaphoreType.DMA,
        pltpu.SemaphoreType.DMA,
    ],
)
def _gather_sc(ids_hbm, tok_tbl_hbm, out_hbm,
               idx_all, tok0, tok1, gsem0, gsem1, osem0, osem1):
    wid = lax.axis_index("s") * NC + lax.axis_index("c")
    base = wid * TOK_PER_W
    toks = (tok0, tok1)
    gsems = (gsem0, gsem1)
    osems = (osem0, osem1)

    pltpu.sync_copy(ids_hbm.at[pl.ds(base, TOK_PER_W)],
                    idx_all.at[pl.ds(0, TOK_PER_W)])
    for p in range(C // 16):
        idx_all[pl.ds(TOK_PER_W + p * 16, 16)] = jnp.zeros((16,), jnp.int32)

    def start_gather(c, b):
        pltpu.async_copy(tok_tbl_hbm.at[idx_all.at[pl.ds(c * C, C)]],
                         toks[b], gsems[b])

    def wait_gather(b):
        pltpu.make_async_copy(tok_tbl_hbm.at[pl.ds(0, C)], toks[b],
                              gsems[b]).wait()

    def start_out(c, b):
        pltpu.async_copy(toks[b], out_hbm.at[pl.ds(base + c * C, C)],
                         osems[b])

    def wait_out(b):
        pltpu.make_async_copy(toks[b], out_hbm.at[pl.ds(0, C)],
                              osems[b]).wait()

    start_gather(0, 0)

    def _phase(c, carry):
        par = lax.rem(c, 2)
        even = par == 0

        def body(b):
            nb = 1 - b
            @pl.when(c > 0)
            def _():
                wait_out(nb)

            start_gather(c + 1, nb)
            wait_gather(b)
            start_out(c, b)

        @pl.when(even)
        def _():
            body(0)

        @pl.when(jnp.logical_not(even))
        def _():
            body(1)

        return carry

    lax.fori_loop(0, NCHUNK, _phase, 0)

    wait_gather(0)
    wait_out(1)


BLK = 2048


def _add_body(cids_ref, tok_ref, ctbl_ref, out_ref):
    cid = cids_ref[0, 0, :]
    onehot = (cid[:, None] == lax.iota(jnp.int32, NT)[None, :])
    code = jnp.dot(onehot.astype(jnp.float32), ctbl_ref[...],
                   preferred_element_type=jnp.float32)
    out_ref[...] = tok_ref[...] + code


_add_tc = pl.pallas_call(
    _add_body,
    grid=(NH // BLK,),
    in_specs=[
        pl.BlockSpec((1, 1, BLK), lambda i: (i, 0, 0)),
        pl.BlockSpec((BLK, D), lambda i: (i, 0)),
        pl.BlockSpec((NT, D), lambda i: (0, 0)),
    ],
    out_specs=pl.BlockSpec((BLK, D), lambda i: (i, 0)),
    out_shape=jax.ShapeDtypeStruct((NH, D), jnp.float32),
)


def kernel(input_ids, code_type_ids, token_table, code_table):
    ids = input_ids.reshape(N).astype(jnp.int32)
    cids = code_type_ids.reshape(N // BLK, 1, BLK).astype(jnp.int32)
    g = _gather_sc(ids, token_table)
    return _add_tc(cids, g, code_table).reshape(B, S, D)
